# baseline (speedup 1.0000x reference)
import jax
import jax.numpy as jnp
from jax.experimental import pallas as pl
from jax.experimental.pallas import tpu as pltpu


def _pos_kernel(row_ref, col_ref, out_ref, scratch):
    i = pl.program_id(0)
    f = col_ref.shape[1]
    h = row_ref.shape[0]
    w = col_ref.shape[0]

    @pl.when(i == 0)
    def _():
        tcol = jnp.transpose(col_ref[...], (1, 0))
        scratch[0:f, :] = jnp.broadcast_to(tcol[:, None, :], (f, h, w)).reshape(
            f, h * w
        )
        trow = jnp.transpose(row_ref[...], (1, 0))
        scratch[f : 2 * f, :] = jnp.broadcast_to(trow[:, :, None], (f, h, w)).reshape(
            f, h * w
        )

    out_ref[0] = scratch[...]


def kernel(mask, row_embed, col_embed):
    b = mask.shape[0]
    h, w = mask.shape[-2], mask.shape[-1]
    f = col_embed.shape[-1]

    out = pl.pallas_call(
        _pos_kernel,
        grid=(b,),
        in_specs=[
            pl.BlockSpec((h, f), lambda i: (0, 0)),
            pl.BlockSpec((w, f), lambda i: (0, 0)),
        ],
        out_specs=pl.BlockSpec((1, 2 * f, h * w), lambda i: (i, 0, 0)),
        out_shape=jax.ShapeDtypeStruct((b, 2 * f, h * w), jnp.float32),
        scratch_shapes=[pltpu.VMEM((2 * f, h * w), jnp.float32)],
    )(row_embed, col_embed)
    return out.reshape(b, 2 * f, h, w)


# speedup vs baseline: 1.0001x; 1.0001x over previous
import jax
import jax.numpy as jnp
from jax.experimental import pallas as pl
from jax.experimental.pallas import tpu as pltpu


def _pos_kernel(row_ref, col_ref, out_ref, scratch, sems):
    b = out_ref.shape[0]
    f = col_ref.shape[1]
    h = row_ref.shape[0]
    w = col_ref.shape[0]

    tcol = jnp.transpose(col_ref[...], (1, 0))
    scratch[0:f, :] = jnp.broadcast_to(tcol[:, None, :], (f, h, w)).reshape(f, h * w)
    trow = jnp.transpose(row_ref[...], (1, 0))
    scratch[f : 2 * f, :] = jnp.broadcast_to(trow[:, :, None], (f, h, w)).reshape(
        f, h * w
    )

    for i in range(b):
        pltpu.make_async_copy(scratch, out_ref.at[i], sems.at[i]).start(
            priority=i % 2
        )
    for i in range(b):
        pltpu.make_async_copy(scratch, out_ref.at[i], sems.at[i]).wait()


def kernel(mask, row_embed, col_embed):
    b = mask.shape[0]
    h, w = mask.shape[-2], mask.shape[-1]
    f = col_embed.shape[-1]

    out = pl.pallas_call(
        _pos_kernel,
        in_specs=[
            pl.BlockSpec(memory_space=pltpu.MemorySpace.VMEM),
            pl.BlockSpec(memory_space=pltpu.MemorySpace.VMEM),
        ],
        out_specs=pl.BlockSpec(memory_space=pltpu.MemorySpace.HBM),
        out_shape=jax.ShapeDtypeStruct((b, 2 * f, h * w), jnp.float32),
        scratch_shapes=[
            pltpu.VMEM((2 * f, h * w), jnp.float32),
            pltpu.SemaphoreType.DMA((b,)),
        ],
    )(row_embed, col_embed)
    return out.reshape(b, 2 * f, h, w)


# speedup vs baseline: 1.0028x; 1.0028x over previous
import jax
import jax.numpy as jnp
from jax.experimental import pallas as pl
from jax.experimental.pallas import tpu as pltpu

_CHUNKS = 4


def _pos_kernel(row_ref, col_ref, out_ref, scratch, sems):
    b = out_ref.shape[0]
    f = col_ref.shape[1]
    h = row_ref.shape[0]
    w = col_ref.shape[0]

    tcol = jnp.transpose(col_ref[...], (1, 0))
    scratch[0:f, :] = jnp.broadcast_to(tcol[:, None, :], (f, h, w)).reshape(f, h * w)
    trow = jnp.transpose(row_ref[...], (1, 0))
    scratch[f : 2 * f, :] = jnp.broadcast_to(trow[:, :, None], (f, h, w)).reshape(
        f, h * w
    )

    rows = 2 * f // _CHUNKS
    for i in range(b):
        for j in range(_CHUNKS):
            pltpu.make_async_copy(
                scratch.at[pl.ds(j * rows, rows)],
                out_ref.at[i, pl.ds(j * rows, rows)],
                sems.at[i, j],
            ).start()
    for i in range(b):
        for j in range(_CHUNKS):
            pltpu.make_async_copy(
                scratch.at[pl.ds(j * rows, rows)],
                out_ref.at[i, pl.ds(j * rows, rows)],
                sems.at[i, j],
            ).wait()


def kernel(mask, row_embed, col_embed):
    b = mask.shape[0]
    h, w = mask.shape[-2], mask.shape[-1]
    f = col_embed.shape[-1]

    out = pl.pallas_call(
        _pos_kernel,
        in_specs=[
            pl.BlockSpec(memory_space=pltpu.MemorySpace.VMEM),
            pl.BlockSpec(memory_space=pltpu.MemorySpace.VMEM),
        ],
        out_specs=pl.BlockSpec(memory_space=pltpu.MemorySpace.HBM),
        out_shape=jax.ShapeDtypeStruct((b, 2 * f, h * w), jnp.float32),
        scratch_shapes=[
            pltpu.VMEM((2 * f, h * w), jnp.float32),
            pltpu.SemaphoreType.DMA((b, _CHUNKS)),
        ],
    )(row_embed, col_embed)
    return out.reshape(b, 2 * f, h, w)
